# baseline (speedup 1.0000x reference)
import functools

import jax
import jax.numpy as jnp
from jax import lax
from jax.experimental import pallas as pl
from jax.experimental.pallas import tpu as pltpu
from jax.experimental.pallas import tpu_sc as plsc

D_MODEL = 4096

_NC = 2
_NS = 16
_NW = _NC * _NS
_CHUNK = 16


def _sc_gather(table, ids3):
    n_ch = ids3.shape[1]
    rows_per_w = n_ch * _CHUNK
    total = _NW * rows_per_w
    d = table.shape[1]
    mesh = plsc.VectorSubcoreMesh(core_axis_name="c", subcore_axis_name="s")

    @functools.partial(
        pl.kernel,
        mesh=mesh,
        out_type=jax.ShapeDtypeStruct((total, d), jnp.float32),
        scratch_types=[
            pltpu.VMEM((n_ch, _CHUNK), jnp.int32),
            pltpu.VMEM((_CHUNK, d), jnp.float32),
            pltpu.SemaphoreType.DMA,
        ],
    )
    def gather_kernel(table_hbm, ids_hbm, out_hbm, idx_v, rows_v, sem):
        wid = lax.axis_index("s") * _NC + lax.axis_index("c")
        base = wid * rows_per_w
        pltpu.sync_copy(ids_hbm.at[wid], idx_v)

        def body(c, _):
            pltpu.async_copy(table_hbm.at[idx_v.at[c]], rows_v, sem).wait()
            pltpu.sync_copy(rows_v, out_hbm.at[pl.ds(base + c * _CHUNK, _CHUNK)])
            return _

        lax.fori_loop(0, n_ch, body, None)

    return gather_kernel(table, ids3)


def _tc_linear_slab(x, w_bf16, b2, s, batch, out_prev):
    n, d = x.shape
    bt, bo = 512, 512
    n_t, n_o = n // bt, d // bo

    def body(*refs):
        x_ref, w_ref, b_ref = refs[0], refs[1], refs[2]
        o_ref, xb = refs[-2], refs[-1]

        @pl.when(pl.program_id(1) == 0)
        def _():
            xb[...] = x_ref[...].astype(jnp.bfloat16)

        acc = lax.dot_general(
            xb[...], w_ref[...], (((1,), (1,)), ((), ())),
            preferred_element_type=jnp.float32,
        )
        o_ref[0] = acc + b_ref[...]

    in_specs = [
        pl.BlockSpec((bt, d), lambda t, o: (t, 0)),
        pl.BlockSpec((bo, d), lambda t, o: (o, 0)),
        pl.BlockSpec((1, bo), lambda t, o: (0, o)),
    ]
    args = [x, w_bf16, b2]
    aliases = {}
    if out_prev is not None:
        in_specs.append(pl.BlockSpec(memory_space=pl.ANY))
        args.append(out_prev)
        aliases = {3: 0}
    return pl.pallas_call(
        body,
        grid=(n_t, n_o),
        in_specs=in_specs,
        out_specs=pl.BlockSpec((1, bt, bo), lambda t, o: (s, t, o)),
        out_shape=jax.ShapeDtypeStruct((batch, n, d), jnp.float32),
        scratch_shapes=[pltpu.VMEM((bt, d), jnp.bfloat16)],
        input_output_aliases=aliases,
    )(*args)


def kernel(input_ids, embed_table, W, b):
    batch, seq = input_ids.shape
    w_bf = W.astype(jnp.bfloat16)
    b2 = b.reshape(1, -1)
    ids = input_ids.astype(jnp.int32)
    xs = []
    for s in range(batch):
        ids3 = ids[s].reshape(_NW, seq // (_NW * _CHUNK), _CHUNK)
        xs.append(_sc_gather(embed_table, ids3))
    out = None
    for s in range(batch):
        out = _tc_linear_slab(xs[s], w_bf, b2, s, batch, out)
    return out


# speedup vs baseline: 1.1909x; 1.1909x over previous
import functools

import jax
import jax.numpy as jnp
from jax import lax
from jax.experimental import pallas as pl
from jax.experimental.pallas import tpu as pltpu
from jax.experimental.pallas import tpu_sc as plsc

D_MODEL = 4096

_NC = 2
_NS = 16
_NW = _NC * _NS
_CHUNK = 16


def _sc_gather(table, ids3):
    n_ch = ids3.shape[1]
    rows_per_w = n_ch * _CHUNK
    total = _NW * rows_per_w
    d = table.shape[1]
    mesh = plsc.VectorSubcoreMesh(core_axis_name="c", subcore_axis_name="s")

    @functools.partial(
        pl.kernel,
        mesh=mesh,
        out_type=jax.ShapeDtypeStruct((total, d), jnp.float32),
        scratch_types=[
            pltpu.VMEM((n_ch, _CHUNK), jnp.int32),
            pltpu.VMEM((_CHUNK, d), jnp.float32),
            pltpu.SemaphoreType.DMA,
        ],
    )
    def gather_kernel(table_hbm, ids_hbm, out_hbm, idx_v, rows_v, sem):
        wid = lax.axis_index("s") * _NC + lax.axis_index("c")
        base = wid * rows_per_w
        pltpu.sync_copy(ids_hbm.at[wid], idx_v)

        def body(c, _):
            pltpu.async_copy(table_hbm.at[idx_v.at[c]], rows_v, sem).wait()
            pltpu.sync_copy(rows_v, out_hbm.at[pl.ds(base + c * _CHUNK, _CHUNK)])
            return _

        lax.fori_loop(0, n_ch, body, None)

    return gather_kernel(table, ids3)


def _tc_linear_slab(x, w_bf16, b2, s, batch, out_prev):
    n, d = x.shape
    bt, bo = 512, 1024
    n_t, n_o = n // bt, d // bo

    def body(*refs):
        x_ref, w_ref, b_ref = refs[0], refs[1], refs[2]
        o_ref, xb = refs[-2], refs[-1]
        o = pl.program_id(1)

        @pl.when(o == 0)
        def _():
            xb[...] = x_ref[...].astype(jnp.bfloat16)

        acc = lax.dot_general(
            xb[...], w_ref[pl.ds(o * bo, bo), :], (((1,), (1,)), ((), ())),
            preferred_element_type=jnp.float32,
        )
        o_ref[0] = acc + b_ref[...]

    in_specs = [
        pl.BlockSpec((bt, d), lambda t, o: (t, 0)),
        pl.BlockSpec((d, d), lambda t, o: (0, 0)),
        pl.BlockSpec((1, bo), lambda t, o: (0, o)),
    ]
    args = [x, w_bf16, b2]
    aliases = {}
    if out_prev is not None:
        in_specs.append(pl.BlockSpec(memory_space=pl.ANY))
        args.append(out_prev)
        aliases = {3: 0}
    return pl.pallas_call(
        body,
        grid=(n_t, n_o),
        in_specs=in_specs,
        out_specs=pl.BlockSpec((1, bt, bo), lambda t, o: (s, t, o)),
        out_shape=jax.ShapeDtypeStruct((batch, n, d), jnp.float32),
        scratch_shapes=[pltpu.VMEM((bt, d), jnp.bfloat16)],
        input_output_aliases=aliases,
        compiler_params=pltpu.CompilerParams(vmem_limit_bytes=110 * 1024 * 1024),
    )(*args)


def kernel(input_ids, embed_table, W, b):
    batch, seq = input_ids.shape
    w_bf = W.astype(jnp.bfloat16)
    b2 = b.reshape(1, -1)
    ids = input_ids.astype(jnp.int32)
    xs = []
    for s in range(batch):
        ids3 = ids[s].reshape(_NW, seq // (_NW * _CHUNK), _CHUNK)
        xs.append(_sc_gather(embed_table, ids3))
    out = None
    for s in range(batch):
        out = _tc_linear_slab(xs[s], w_bf, b2, s, batch, out)
    return out


# speedup vs baseline: 1.2292x; 1.0322x over previous
import functools

import jax
import jax.numpy as jnp
from jax import lax
from jax.experimental import pallas as pl
from jax.experimental.pallas import tpu as pltpu
from jax.experimental.pallas import tpu_sc as plsc

D_MODEL = 4096

_NC = 2
_NS = 16
_NW = _NC * _NS
_CHUNK = 16


def _sc_gather(table, ids3):
    n_ch = ids3.shape[1]
    rows_per_w = n_ch * _CHUNK
    total = _NW * rows_per_w
    d = table.shape[1]
    mesh = plsc.VectorSubcoreMesh(core_axis_name="c", subcore_axis_name="s")

    @functools.partial(
        pl.kernel,
        mesh=mesh,
        out_type=jax.ShapeDtypeStruct((total, d), jnp.float32),
        scratch_types=[
            pltpu.VMEM((n_ch, _CHUNK), jnp.int32),
            pltpu.VMEM((_CHUNK, d), jnp.float32),
            pltpu.SemaphoreType.DMA,
        ],
    )
    def gather_kernel(table_hbm, ids_hbm, out_hbm, idx_v, rows_v, sem):
        wid = lax.axis_index("s") * _NC + lax.axis_index("c")
        base = wid * rows_per_w
        pltpu.sync_copy(ids_hbm.at[wid], idx_v)

        def body(c, _):
            pltpu.async_copy(table_hbm.at[idx_v.at[c]], rows_v, sem).wait()
            pltpu.sync_copy(rows_v, out_hbm.at[pl.ds(base + c * _CHUNK, _CHUNK)])
            return _

        lax.fori_loop(0, n_ch, body, None)

    return gather_kernel(table, ids3)


def _tc_linear_slab(x, w_bf16, b2, s, batch, out_prev):
    n, d = x.shape
    bt, bo = 512, 2048
    n_t, n_o = n // bt, d // bo

    def body(*refs):
        x_ref, w_ref, b_ref = refs[0], refs[1], refs[2]
        o_ref, xb = refs[-2], refs[-1]
        o = pl.program_id(1)

        @pl.when(o == 0)
        def _():
            xb[...] = x_ref[...].astype(jnp.bfloat16)

        acc = lax.dot_general(
            xb[...], w_ref[pl.ds(o * bo, bo), :], (((1,), (1,)), ((), ())),
            preferred_element_type=jnp.float32,
        )
        o_ref[0] = acc + b_ref[...]

    in_specs = [
        pl.BlockSpec((bt, d), lambda t, o: (t, 0)),
        pl.BlockSpec((d, d), lambda t, o: (0, 0)),
        pl.BlockSpec((1, bo), lambda t, o: (0, o)),
    ]
    args = [x, w_bf16, b2]
    aliases = {}
    if out_prev is not None:
        in_specs.append(pl.BlockSpec(memory_space=pl.ANY))
        args.append(out_prev)
        aliases = {3: 0}
    return pl.pallas_call(
        body,
        grid=(n_t, n_o),
        in_specs=in_specs,
        out_specs=pl.BlockSpec((1, bt, bo), lambda t, o: (s, t, o)),
        out_shape=jax.ShapeDtypeStruct((batch, n, d), jnp.float32),
        scratch_shapes=[pltpu.VMEM((bt, d), jnp.bfloat16)],
        input_output_aliases=aliases,
        compiler_params=pltpu.CompilerParams(vmem_limit_bytes=110 * 1024 * 1024),
    )(*args)


def kernel(input_ids, embed_table, W, b):
    batch, seq = input_ids.shape
    w_bf = W.astype(jnp.bfloat16)
    b2 = b.reshape(1, -1)
    ids = input_ids.astype(jnp.int32)
    xs = []
    for s in range(batch):
        ids3 = ids[s].reshape(_NW, seq // (_NW * _CHUNK), _CHUNK)
        xs.append(_sc_gather(embed_table, ids3))
    out = None
    for s in range(batch):
        out = _tc_linear_slab(xs[s], w_bf, b2, s, batch, out)
    return out


# speedup vs baseline: 1.2748x; 1.0371x over previous
import functools

import jax
import jax.numpy as jnp
from jax import lax
from jax.experimental import pallas as pl
from jax.experimental.pallas import tpu as pltpu
from jax.experimental.pallas import tpu_sc as plsc

D_MODEL = 4096

_NC = 2
_NS = 16
_NW = _NC * _NS
_CHUNK = 16


def _sc_gather(table, ids3):
    n_ch = ids3.shape[1]
    rows_per_w = n_ch * _CHUNK
    total = _NW * rows_per_w
    d = table.shape[1]
    mesh = plsc.VectorSubcoreMesh(core_axis_name="c", subcore_axis_name="s")

    @functools.partial(
        pl.kernel,
        mesh=mesh,
        out_type=jax.ShapeDtypeStruct((total, d), jnp.float32),
        scratch_types=[
            pltpu.VMEM((n_ch, _CHUNK), jnp.int32),
            pltpu.VMEM((_CHUNK, d), jnp.float32),
            pltpu.SemaphoreType.DMA,
        ],
    )
    def gather_kernel(table_hbm, ids_hbm, out_hbm, idx_v, rows_v, sem):
        wid = lax.axis_index("s") * _NC + lax.axis_index("c")
        base = wid * rows_per_w
        pltpu.sync_copy(ids_hbm.at[wid], idx_v)

        def body(c, _):
            pltpu.async_copy(table_hbm.at[idx_v.at[c]], rows_v, sem).wait()
            pltpu.sync_copy(rows_v, out_hbm.at[pl.ds(base + c * _CHUNK, _CHUNK)])
            return _

        lax.fori_loop(0, n_ch, body, None)

    return gather_kernel(table, ids3)


def _tc_linear_slab(x, w_bf16, b2, s, batch, out_prev):
    n, d = x.shape
    bt = 256
    n_t = n // bt

    def body(*refs):
        x_ref, w_ref, b_ref = refs[0], refs[1], refs[2]
        o_ref = refs[-1]
        acc = lax.dot_general(
            x_ref[...].astype(jnp.bfloat16), w_ref[...], (((1,), (1,)), ((), ())),
            preferred_element_type=jnp.float32,
        )
        o_ref[0] = acc + b_ref[...]

    in_specs = [
        pl.BlockSpec((bt, d), lambda t: (t, 0)),
        pl.BlockSpec((d, d), lambda t: (0, 0)),
        pl.BlockSpec((1, d), lambda t: (0, 0)),
    ]
    args = [x, w_bf16, b2]
    aliases = {}
    if out_prev is not None:
        in_specs.append(pl.BlockSpec(memory_space=pl.ANY))
        args.append(out_prev)
        aliases = {3: 0}
    return pl.pallas_call(
        body,
        grid=(n_t,),
        in_specs=in_specs,
        out_specs=pl.BlockSpec((1, bt, d), lambda t: (s, t, 0)),
        out_shape=jax.ShapeDtypeStruct((batch, n, d), jnp.float32),
        input_output_aliases=aliases,
        compiler_params=pltpu.CompilerParams(vmem_limit_bytes=110 * 1024 * 1024),
    )(*args)


def kernel(input_ids, embed_table, W, b):
    batch, seq = input_ids.shape
    w_bf = W.astype(jnp.bfloat16)
    b2 = b.reshape(1, -1)
    ids = input_ids.astype(jnp.int32)
    xs = []
    for s in range(batch):
        ids3 = ids[s].reshape(_NW, seq // (_NW * _CHUNK), _CHUNK)
        xs.append(_sc_gather(embed_table, ids3))
    out = None
    for s in range(batch):
        out = _tc_linear_slab(xs[s], w_bf, b2, s, batch, out)
    return out
